# baseline (speedup 1.0000x reference)
import jax
import jax.numpy as jnp
from jax.experimental import pallas as pl

_B = 16
_Q = 20000
_G = 4
_SCALE = 1.0 / 512.0


def _body(in_ref, out_ref):
    g = pl.program_id(0)
    k = pl.program_id(1)
    v = in_ref[0]
    s = v * _SCALE
    mn = jnp.clip(jnp.minimum(s[1:3, :], s[3:5, :]), 0.0, 1.0)
    mx = jnp.clip(jnp.maximum(s[1:3, :], s[3:5, :]), 0.0, 1.0)
    bf = (g * _G + k).astype(jnp.float32)
    brow = jnp.zeros((1, _Q), jnp.float32) + bf
    res = jnp.concatenate([brow, mn, mx], axis=0)
    for jj in range(_G):
        @pl.when(k == jj)
        def _():
            out_ref[:, jj * _Q:(jj + 1) * _Q] = res


def kernel(vit_output, input_images_or_features):
    del input_images_or_features
    vt = jnp.transpose(vit_output, (0, 2, 1))
    out = pl.pallas_call(
        _body,
        grid=(_B // _G, _G),
        in_specs=[pl.BlockSpec((1, 8, _Q), lambda g, k: (g * _G + k, 0, 0))],
        out_specs=pl.BlockSpec((5, _G * _Q), lambda g, k: (0, g)),
        out_shape=jax.ShapeDtypeStruct((5, _B * _Q), jnp.float32),
    )(vt)
    return out.T


# speedup vs baseline: 1.8724x; 1.8724x over previous
import jax
import jax.numpy as jnp
from jax.experimental import pallas as pl

_B = 16
_Q = 20000
_G = 4
_SCALE = 1.0 / 512.0


def _body(in_ref, out_ref):
    g = pl.program_id(0)
    for j in range(_G):
        v = in_ref[j]
        s = v * _SCALE
        mn = jnp.clip(jnp.minimum(s[1:3, :], s[3:5, :]), 0.0, 1.0)
        mx = jnp.clip(jnp.maximum(s[1:3, :], s[3:5, :]), 0.0, 1.0)
        bf = (g * _G + j).astype(jnp.float32)
        brow = jnp.zeros((1, _Q), jnp.float32) + bf
        res = jnp.concatenate([brow, mn, mx], axis=0)
        out_ref[:, j * _Q:(j + 1) * _Q] = res


def kernel(vit_output, input_images_or_features):
    del input_images_or_features
    vt = jnp.transpose(vit_output, (0, 2, 1))
    out = pl.pallas_call(
        _body,
        grid=(_B // _G,),
        in_specs=[pl.BlockSpec((_G, 8, _Q), lambda g: (g, 0, 0))],
        out_specs=pl.BlockSpec((5, _G * _Q), lambda g: (0, g)),
        out_shape=jax.ShapeDtypeStruct((5, _B * _Q), jnp.float32),
    )(vt)
    return out.T
